# baseline (speedup 1.0000x reference)
import functools

import jax
import jax.numpy as jnp
from jax import lax
from jax.experimental import pallas as pl
from jax.experimental.pallas import tpu as pltpu
from jax.experimental.pallas import tpu_sc as plsc

N = 10000
E = 320000
IN_F = 128
D = 64
NW = 32
EPW = E // NW
CHUNK = 80
NCHUNK = EPW // CHUNK
RPT = N // 16
MB = 1000


def _l0_body(x_ref, w_ref, al_ref, ar_ref, ft_ref, el_ref, er_ref):
    x = x_ref[...]
    w = w_ref[...]
    for hd in range(8):
        ft_ref[hd] = jnp.dot(x, w[:, hd * D:(hd + 1) * D],
                             preferred_element_type=jnp.float32)
    w3 = w.reshape(IN_F, 8, D)
    wal = (w3 * al_ref[...][None]).sum(-1)
    war = (w3 * ar_ref[...][None]).sum(-1)
    el_ref[...] = jnp.dot(x, wal, preferred_element_type=jnp.float32)
    er_ref[...] = jnp.dot(x, war, preferred_element_type=jnp.float32)


def _mid_body(out_ref, den_ref, w1_ref, al1_ref, ar1_ref, b0_ref,
              ft_ref, el_ref, er_ref):
    dnm = den_ref[0] + den_ref[1]
    acc = jnp.zeros((MB, D), jnp.float32)
    for hd in range(8):
        oh = out_ref[0, hd] + out_ref[1, hd]
        ohn = oh / (dnm[:, hd:hd + 1] + 1e-9) + b0_ref[hd][None, :]
        hh = jnp.where(ohn > 0, ohn, jnp.exp(ohn) - 1.0)
        acc = acc + jnp.dot(hh, w1_ref[hd * D:(hd + 1) * D, :],
                            preferred_element_type=jnp.float32)
    ft_ref[...] = acc
    el_ref[...] = (acc * al1_ref[...]).sum(-1, keepdims=True)
    er_ref[...] = (acc * ar1_ref[...]).sum(-1, keepdims=True)


def _fin_body(out_ref, den_ref, b1_ref, o_ref):
    dnm = den_ref[0] + den_ref[1]
    o = out_ref[0] + out_ref[1]
    o_ref[...] = o / (dnm + 1e-9) + b1_ref[...]


NBLK = N // CHUNK


def _make_sc_body(n_heads):
    def body(ftH, elT, erT, srcr, dstr, znd, zn, out_hbm, den_hbm,
             el_v, er_v, src_v, dst_v, srcb, wbuf, rows_v, out_s, den_s,
             semg, semd, semo):
        c = lax.axis_index("c")
        s = lax.axis_index("s")
        gid = c * 16 + s
        pltpu.sync_copy(srcr.at[gid], src_v)
        pltpu.sync_copy(dstr.at[gid], dst_v)

        def per_head(h, carry):
            pltpu.sync_copy(elT.at[h, 0], el_v)
            pltpu.sync_copy(erT.at[h, 0], er_v)
            for it in range(8):
                blk = it * 16 + s

                @pl.when(blk < NBLK)
                def _(blk=blk):
                    off = pl.multiple_of(blk * CHUNK, CHUNK)
                    pltpu.sync_copy(znd.at[pl.ds(off, CHUNK)],
                                    out_s.at[pl.ds(off, CHUNK)])

            @pl.when(s == 0)
            def _():
                pltpu.sync_copy(zn, den_s)

            plsc.subcore_barrier()

            def fire_gather(jn, p, guard_rows):
                if guard_rows:
                    @pl.when(jn >= 4)
                    def _():
                        pltpu.make_async_copy(
                            rows_v.at[p], out_s.at[dst_v.at[jn]],
                            semo.at[p]).wait()
                for q in range(CHUNK // 16):
                    srcb[p, pl.ds(q * 16, 16)] = (
                        src_v[jn, pl.ds(q * 16, 16)] + h * N)
                pltpu.async_copy(ftH.at[srcb.at[p]], rows_v.at[p],
                                 semg.at[p])

            def consume(j, p, guard_w):
                woff = p * CHUNK
                if guard_w:
                    @pl.when(j >= 4)
                    def _():
                        pltpu.make_async_copy(
                            wbuf.at[pl.ds(woff, CHUNK)],
                            den_s.at[dst_v.at[j]], semd.at[p]).wait()
                for q in range(CHUNK // 16):
                    si = src_v[j, pl.ds(q * 16, 16)]
                    di = dst_v[j, pl.ds(q * 16, 16)]
                    t = (plsc.load_gather(el_v, [si])
                         + plsc.load_gather(er_v, [di]))
                    t = jnp.where(t >= 0, t, 0.2 * t)
                    wbuf[pl.ds(woff + q * 16, 16)] = jnp.exp(t)
                pltpu.async_copy(wbuf.at[pl.ds(woff, CHUNK)],
                                 den_s.at[dst_v.at[j]], semd.at[p],
                                 add=True)
                pltpu.make_async_copy(ftH.at[srcb.at[p]], rows_v.at[p],
                                      semg.at[p]).wait()

                @plsc.parallel_loop(0, CHUNK, unroll=16)
                def _(e):
                    wv = plsc.load_gather(
                        wbuf, [jnp.full((16,), woff + e, jnp.int32)])
                    for q in range(D // 16):
                        rows_v[p, e, pl.ds(q * 16, 16)] = (
                            rows_v[p, e, pl.ds(q * 16, 16)] * wv)
                pltpu.async_copy(rows_v.at[p], out_s.at[dst_v.at[j]],
                                 semo.at[p], add=True)

            fire_gather(0, 0, False)
            fire_gather(1, 1, False)

            def quad(jj, carry_c):
                for b in range(4):
                    j = 4 * jj + b
                    consume(j, b, True)
                    jn = j + 2
                    pn = (b + 2) % 4

                    @pl.when(jn < NCHUNK)
                    def _(jn=jn, pn=pn):
                        fire_gather(jn, pn, True)

                return carry_c

            lax.fori_loop(0, NCHUNK // 4, quad, 0)
            consume(jnp.int32(NCHUNK - 1), 0, True)

            for p2 in range(4):
                pltpu.make_async_copy(
                    wbuf.at[pl.ds(p2 * CHUNK, CHUNK)],
                    den_s.at[dst_v.at[0]], semd.at[p2]).wait()
                pltpu.make_async_copy(
                    rows_v.at[p2], out_s.at[dst_v.at[0]],
                    semo.at[p2]).wait()

            plsc.subcore_barrier()
            for it in range(8):
                blk = it * 16 + s

                @pl.when(blk < NBLK)
                def _(blk=blk):
                    off = pl.multiple_of(blk * CHUNK, CHUNK)
                    pltpu.sync_copy(out_s.at[pl.ds(off, CHUNK)],
                                    out_hbm.at[c].at[h].at[pl.ds(off, CHUNK)])

            @pl.when(s == 0)
            def _():
                pltpu.sync_copy(den_s, den_hbm.at[c, h, 0])

            plsc.subcore_barrier()
            return carry

        lax.fori_loop(0, n_heads, per_head, 0)

    return body


def _sc_edge(n_heads, ftH, elT, erT, srcr, dstr):
    znd = jnp.zeros((N, D), jnp.float32)
    zn = jnp.zeros((N,), jnp.float32)
    fn = pl.kernel(
        _make_sc_body(n_heads),
        out_type=[jax.ShapeDtypeStruct((2, n_heads, N, D), jnp.float32),
                  jax.ShapeDtypeStruct((2, n_heads, 1, N), jnp.float32)],
        mesh=plsc.VectorSubcoreMesh(core_axis_name="c", subcore_axis_name="s"),
        compiler_params=pltpu.CompilerParams(needs_layout_passes=False,
                                             use_tc_tiling_on_sc=False),
        scratch_types=[
            pltpu.VMEM((N,), jnp.float32),
            pltpu.VMEM((N,), jnp.float32),
            pltpu.VMEM((NCHUNK, CHUNK), jnp.int32),
            pltpu.VMEM((NCHUNK, CHUNK), jnp.int32),
            pltpu.VMEM((4, CHUNK), jnp.int32),
            pltpu.VMEM((4 * CHUNK,), jnp.float32),
            pltpu.VMEM((4, CHUNK, D), jnp.float32),
            pltpu.VMEM_SHARED((N, D), jnp.float32),
            pltpu.VMEM_SHARED((N,), jnp.float32),
            pltpu.SemaphoreType.DMA((4,)),
            pltpu.SemaphoreType.DMA((4,)),
            pltpu.SemaphoreType.DMA((4,)),
        ],
    )
    return fn(ftH, elT, erT, srcr, dstr, znd, zn)


def kernel(features, edge_index, W0, al0, ar0, b0, W1, al1, ar1, b1):
    src = edge_index[0].astype(jnp.int32)
    dst = edge_index[1].astype(jnp.int32)
    srcr = src.reshape(NW, NCHUNK, CHUNK)
    dstr = dst.reshape(NW, NCHUNK, CHUNK)

    ft0, el0, er0 = pl.pallas_call(
        _l0_body, grid=(N // MB,),
        in_specs=[pl.BlockSpec((MB, IN_F), lambda i: (i, 0)),
                  pl.BlockSpec((IN_F, 8 * D), lambda i: (0, 0)),
                  pl.BlockSpec((8, D), lambda i: (0, 0)),
                  pl.BlockSpec((8, D), lambda i: (0, 0))],
        out_specs=[pl.BlockSpec((8, MB, D), lambda i: (0, i, 0)),
                   pl.BlockSpec((MB, 8), lambda i: (i, 0)),
                   pl.BlockSpec((MB, 8), lambda i: (i, 0))],
        out_shape=[jax.ShapeDtypeStruct((8, N, D), jnp.float32),
                   jax.ShapeDtypeStruct((N, 8), jnp.float32),
                   jax.ShapeDtypeStruct((N, 8), jnp.float32)],
    )(features, W0, al0, ar0)

    ftH0 = ft0.reshape(8 * N, D)
    out0, den0 = _sc_edge(8, ftH0,
                          jnp.transpose(el0).reshape(8, 1, N),
                          jnp.transpose(er0).reshape(8, 1, N),
                          srcr, dstr)

    den0t = jnp.transpose(den0.reshape(2, 8, N), (0, 2, 1))
    ft1, el1, er1 = pl.pallas_call(
        _mid_body, grid=(N // MB,),
        in_specs=[pl.BlockSpec((2, 8, MB, D), lambda i: (0, 0, i, 0)),
                  pl.BlockSpec((2, MB, 8), lambda i: (0, i, 0)),
                  pl.BlockSpec((8 * D, D), lambda i: (0, 0)),
                  pl.BlockSpec((1, D), lambda i: (0, 0)),
                  pl.BlockSpec((1, D), lambda i: (0, 0)),
                  pl.BlockSpec((8, D), lambda i: (0, 0))],
        out_specs=[pl.BlockSpec((MB, D), lambda i: (i, 0)),
                   pl.BlockSpec((MB, 1), lambda i: (i, 0)),
                   pl.BlockSpec((MB, 1), lambda i: (i, 0))],
        out_shape=[jax.ShapeDtypeStruct((N, D), jnp.float32),
                   jax.ShapeDtypeStruct((N, 1), jnp.float32),
                   jax.ShapeDtypeStruct((N, 1), jnp.float32)],
    )(out0, den0t, W1, al1, ar1, b0.reshape(8, D))

    out1, den1 = _sc_edge(1, ft1,
                          jnp.transpose(el1).reshape(1, 1, N),
                          jnp.transpose(er1).reshape(1, 1, N),
                          srcr, dstr)

    logits = pl.pallas_call(
        _fin_body, grid=(N // MB,),
        in_specs=[pl.BlockSpec((2, MB, D), lambda i: (0, i, 0)),
                  pl.BlockSpec((2, MB, 1), lambda i: (0, i, 0)),
                  pl.BlockSpec((1, D), lambda i: (0, 0))],
        out_specs=pl.BlockSpec((MB, D), lambda i: (i, 0)),
        out_shape=jax.ShapeDtypeStruct((N, D), jnp.float32),
    )(out1.reshape(2, N, D), den1.reshape(2, N, 1), b1.reshape(1, D))
    return logits


# speedup vs baseline: 1.1725x; 1.1725x over previous
import functools

import jax
import jax.numpy as jnp
from jax import lax
from jax.experimental import pallas as pl
from jax.experimental.pallas import tpu as pltpu
from jax.experimental.pallas import tpu_sc as plsc

N = 10000
E = 320000
IN_F = 128
D = 64
NW = 32
EPW = E // NW
CHUNK = 80
NCHUNK = EPW // CHUNK
RPT = N // 16
MB = 1000


def _l0_body(x_ref, w_ref, al_ref, ar_ref, ft_ref, el_ref, er_ref):
    x = x_ref[...]
    w = w_ref[...]
    for hd in range(8):
        ft_ref[hd] = jnp.dot(x, w[:, hd * D:(hd + 1) * D],
                             preferred_element_type=jnp.float32)
    w3 = w.reshape(IN_F, 8, D)
    wal = (w3 * al_ref[...][None]).sum(-1)
    war = (w3 * ar_ref[...][None]).sum(-1)
    el_ref[...] = jnp.dot(x, wal, preferred_element_type=jnp.float32)
    er_ref[...] = jnp.dot(x, war, preferred_element_type=jnp.float32)


def _mid_body(out_ref, den_ref, w1_ref, al1_ref, ar1_ref, b0_ref,
              ft_ref, el_ref, er_ref):
    dnm = den_ref[0] + den_ref[1]
    acc = jnp.zeros((MB, D), jnp.float32)
    for hd in range(8):
        oh = out_ref[0, hd] + out_ref[1, hd]
        ohn = oh / (dnm[:, hd:hd + 1] + 1e-9) + b0_ref[hd][None, :]
        hh = jnp.where(ohn > 0, ohn, jnp.exp(ohn) - 1.0)
        acc = acc + jnp.dot(hh, w1_ref[hd * D:(hd + 1) * D, :],
                            preferred_element_type=jnp.float32)
    ft_ref[...] = acc
    el_ref[...] = (acc * al1_ref[...]).sum(-1, keepdims=True)
    er_ref[...] = (acc * ar1_ref[...]).sum(-1, keepdims=True)


def _fin_body(out_ref, den_ref, b1_ref, o_ref):
    dnm = den_ref[0] + den_ref[1]
    o = out_ref[0] + out_ref[1]
    o_ref[...] = o / (dnm + 1e-9) + b1_ref[...]


NBLK = N // CHUNK


def _make_sc_body(n_heads):
    def body(ftH, elT, erT, srcr, dstr, znd, zn, out_hbm, den_hbm,
             el_v, er_v, src_v, dst_v, srcb, wbuf, rows_v, out_s, den_s,
             semg, semd, semo):
        c = lax.axis_index("c")
        s = lax.axis_index("s")
        gid = c * 16 + s
        pltpu.sync_copy(srcr.at[gid], src_v)
        pltpu.sync_copy(dstr.at[gid], dst_v)

        def per_head(h, carry):
            pltpu.sync_copy(elT.at[h, 0], el_v)
            pltpu.sync_copy(erT.at[h, 0], er_v)
            for it in range(8):
                blk = it * 16 + s

                @pl.when(blk < NBLK)
                def _(blk=blk):
                    off = pl.multiple_of(blk * CHUNK, CHUNK)
                    pltpu.sync_copy(znd.at[pl.ds(off, CHUNK)],
                                    out_s.at[pl.ds(off, CHUNK)])

            @pl.when(s == 0)
            def _():
                pltpu.sync_copy(zn, den_s)

            plsc.subcore_barrier()

            def fire_gather(jn, p, guard_rows):
                if guard_rows:
                    @pl.when(jn >= 4)
                    def _():
                        pltpu.make_async_copy(
                            rows_v.at[p], out_s.at[dst_v.at[jn]],
                            semo.at[p]).wait()
                for q in range(CHUNK // 16):
                    srcb[p, pl.ds(q * 16, 16)] = (
                        src_v[jn, pl.ds(q * 16, 16)] + h * N)
                pltpu.async_copy(ftH.at[srcb.at[p]], rows_v.at[p],
                                 semg.at[p])

            def consume(j, p, guard_w):
                woff = p * CHUNK
                if guard_w:
                    @pl.when(j >= 4)
                    def _():
                        pltpu.make_async_copy(
                            wbuf.at[pl.ds(woff, CHUNK)],
                            den_s.at[dst_v.at[j]], semd.at[p]).wait()
                for q in range(CHUNK // 16):
                    si = src_v[j, pl.ds(q * 16, 16)]
                    di = dst_v[j, pl.ds(q * 16, 16)]
                    t = (plsc.load_gather(el_v, [si])
                         + plsc.load_gather(er_v, [di]))
                    t = jnp.where(t >= 0, t, 0.2 * t)
                    wbuf[pl.ds(woff + q * 16, 16)] = jnp.exp(t)
                pltpu.async_copy(wbuf.at[pl.ds(woff, CHUNK)],
                                 den_s.at[dst_v.at[j]], semd.at[p],
                                 add=True)
                pltpu.make_async_copy(ftH.at[srcb.at[p]], rows_v.at[p],
                                      semg.at[p]).wait()

                @plsc.parallel_loop(0, CHUNK, unroll=8)
                def _(e):
                    wv = plsc.load_gather(
                        wbuf, [jnp.full((16,), woff + e, jnp.int32)])
                    for q in range(D // 16):
                        rows_v[p, e, pl.ds(q * 16, 16)] = (
                            rows_v[p, e, pl.ds(q * 16, 16)] * wv)
                pltpu.async_copy(rows_v.at[p], out_s.at[dst_v.at[j]],
                                 semo.at[p], add=True)

            fire_gather(0, 0, False)
            fire_gather(1, 1, False)
            fire_gather(2, 2, False)

            def quad(jj, carry_c):
                for b in range(4):
                    j = 4 * jj + b
                    consume(j, b, True)
                    jn = j + 3
                    pn = (b + 3) % 4

                    @pl.when(jn < NCHUNK)
                    def _(jn=jn, pn=pn):
                        fire_gather(jn, pn, True)

                return carry_c

            lax.fori_loop(0, NCHUNK // 4, quad, 0)
            consume(jnp.int32(NCHUNK - 1), 0, True)

            for p2 in range(4):
                pltpu.make_async_copy(
                    wbuf.at[pl.ds(p2 * CHUNK, CHUNK)],
                    den_s.at[dst_v.at[0]], semd.at[p2]).wait()
                pltpu.make_async_copy(
                    rows_v.at[p2], out_s.at[dst_v.at[0]],
                    semo.at[p2]).wait()

            plsc.subcore_barrier()
            for it in range(8):
                blk = it * 16 + s

                @pl.when(blk < NBLK)
                def _(blk=blk):
                    off = pl.multiple_of(blk * CHUNK, CHUNK)
                    pltpu.sync_copy(out_s.at[pl.ds(off, CHUNK)],
                                    out_hbm.at[c].at[h].at[pl.ds(off, CHUNK)])

            @pl.when(s == 0)
            def _():
                pltpu.sync_copy(den_s, den_hbm.at[c, h, 0])

            plsc.subcore_barrier()
            return carry

        lax.fori_loop(0, n_heads, per_head, 0)

    return body


def _sc_edge(n_heads, ftH, elT, erT, srcr, dstr):
    znd = jnp.zeros((N, D), jnp.float32)
    zn = jnp.zeros((N,), jnp.float32)
    fn = pl.kernel(
        _make_sc_body(n_heads),
        out_type=[jax.ShapeDtypeStruct((2, n_heads, N, D), jnp.float32),
                  jax.ShapeDtypeStruct((2, n_heads, 1, N), jnp.float32)],
        mesh=plsc.VectorSubcoreMesh(core_axis_name="c", subcore_axis_name="s"),
        compiler_params=pltpu.CompilerParams(needs_layout_passes=False,
                                             use_tc_tiling_on_sc=False),
        scratch_types=[
            pltpu.VMEM((N,), jnp.float32),
            pltpu.VMEM((N,), jnp.float32),
            pltpu.VMEM((NCHUNK, CHUNK), jnp.int32),
            pltpu.VMEM((NCHUNK, CHUNK), jnp.int32),
            pltpu.VMEM((4, CHUNK), jnp.int32),
            pltpu.VMEM((4 * CHUNK,), jnp.float32),
            pltpu.VMEM((4, CHUNK, D), jnp.float32),
            pltpu.VMEM_SHARED((N, D), jnp.float32),
            pltpu.VMEM_SHARED((N,), jnp.float32),
            pltpu.SemaphoreType.DMA((4,)),
            pltpu.SemaphoreType.DMA((4,)),
            pltpu.SemaphoreType.DMA((4,)),
        ],
    )
    return fn(ftH, elT, erT, srcr, dstr, znd, zn)


def kernel(features, edge_index, W0, al0, ar0, b0, W1, al1, ar1, b1):
    src = edge_index[0].astype(jnp.int32)
    dst = edge_index[1].astype(jnp.int32)
    srcr = src.reshape(NW, NCHUNK, CHUNK)
    dstr = dst.reshape(NW, NCHUNK, CHUNK)

    ft0, el0, er0 = pl.pallas_call(
        _l0_body, grid=(N // MB,),
        in_specs=[pl.BlockSpec((MB, IN_F), lambda i: (i, 0)),
                  pl.BlockSpec((IN_F, 8 * D), lambda i: (0, 0)),
                  pl.BlockSpec((8, D), lambda i: (0, 0)),
                  pl.BlockSpec((8, D), lambda i: (0, 0))],
        out_specs=[pl.BlockSpec((8, MB, D), lambda i: (0, i, 0)),
                   pl.BlockSpec((MB, 8), lambda i: (i, 0)),
                   pl.BlockSpec((MB, 8), lambda i: (i, 0))],
        out_shape=[jax.ShapeDtypeStruct((8, N, D), jnp.float32),
                   jax.ShapeDtypeStruct((N, 8), jnp.float32),
                   jax.ShapeDtypeStruct((N, 8), jnp.float32)],
    )(features, W0, al0, ar0)

    ftH0 = ft0.reshape(8 * N, D)
    out0, den0 = _sc_edge(8, ftH0,
                          jnp.transpose(el0).reshape(8, 1, N),
                          jnp.transpose(er0).reshape(8, 1, N),
                          srcr, dstr)

    den0t = jnp.transpose(den0.reshape(2, 8, N), (0, 2, 1))
    ft1, el1, er1 = pl.pallas_call(
        _mid_body, grid=(N // MB,),
        in_specs=[pl.BlockSpec((2, 8, MB, D), lambda i: (0, 0, i, 0)),
                  pl.BlockSpec((2, MB, 8), lambda i: (0, i, 0)),
                  pl.BlockSpec((8 * D, D), lambda i: (0, 0)),
                  pl.BlockSpec((1, D), lambda i: (0, 0)),
                  pl.BlockSpec((1, D), lambda i: (0, 0)),
                  pl.BlockSpec((8, D), lambda i: (0, 0))],
        out_specs=[pl.BlockSpec((MB, D), lambda i: (i, 0)),
                   pl.BlockSpec((MB, 1), lambda i: (i, 0)),
                   pl.BlockSpec((MB, 1), lambda i: (i, 0))],
        out_shape=[jax.ShapeDtypeStruct((N, D), jnp.float32),
                   jax.ShapeDtypeStruct((N, 1), jnp.float32),
                   jax.ShapeDtypeStruct((N, 1), jnp.float32)],
    )(out0, den0t, W1, al1, ar1, b0.reshape(8, D))

    out1, den1 = _sc_edge(1, ft1,
                          jnp.transpose(el1).reshape(1, 1, N),
                          jnp.transpose(er1).reshape(1, 1, N),
                          srcr, dstr)

    logits = pl.pallas_call(
        _fin_body, grid=(N // MB,),
        in_specs=[pl.BlockSpec((2, MB, D), lambda i: (0, i, 0)),
                  pl.BlockSpec((2, MB, 1), lambda i: (0, i, 0)),
                  pl.BlockSpec((1, D), lambda i: (0, 0))],
        out_specs=pl.BlockSpec((MB, D), lambda i: (i, 0)),
        out_shape=jax.ShapeDtypeStruct((N, D), jnp.float32),
    )(out1.reshape(2, N, D), den1.reshape(2, N, 1), b1.reshape(1, D))
    return logits
